# baseline (speedup 1.0000x reference)
import functools
import math

import jax
import jax.numpy as jnp
from jax import lax
from jax.experimental import pallas as pl
from jax.experimental.pallas import tpu as pltpu
from jax.experimental.pallas import tpu_sc as plsc

EMBED = 64
NC, NS = 2, 16
NW = NC * NS
GROUP = 128
K = 8
CHUNK = GROUP * K


@functools.lru_cache(maxsize=None)
def _gather_call(n_total):
    per_w = n_total // NW
    n_chunks = per_w // CHUNK
    mesh = plsc.VectorSubcoreMesh(core_axis_name="c", subcore_axis_name="s")

    @functools.partial(
        pl.kernel,
        out_type=jax.ShapeDtypeStruct((n_total, EMBED), jnp.float32),
        mesh=mesh,
        scratch_types=[
            pltpu.VMEM((K, GROUP), jnp.int32),
            pltpu.VMEM((CHUNK, EMBED), jnp.float32),
            pltpu.SemaphoreType.DMA,
        ],
        compiler_params=pltpu.CompilerParams(use_tc_tiling_on_sc=False),
    )
    def gather(idx_hbm, table_hbm, out_hbm, idx_v, rows_v, sem):
        wid = lax.axis_index("s") * NC + lax.axis_index("c")
        grp0 = wid * (per_w // GROUP)
        row0 = wid * per_w

        @pl.loop(0, n_chunks)
        def _chunk(c):
            pltpu.sync_copy(idx_hbm.at[pl.ds(grp0 + c * K, K)], idx_v)
            copies = [
                pltpu.async_copy(
                    table_hbm.at[idx_v.at[j]],
                    rows_v.at[pl.ds(j * GROUP, GROUP)],
                    sem,
                )
                for j in range(K)
            ]
            for cp in copies:
                cp.wait()
            pltpu.sync_copy(rows_v, out_hbm.at[pl.ds(row0 + c * CHUNK, CHUNK)])

    return gather


def _pe_body(out_ref):
    hist, emb = out_ref.shape
    pos = lax.broadcasted_iota(jnp.int32, (hist, emb), 0).astype(jnp.float32)
    col = lax.broadcasted_iota(jnp.int32, (hist, emb), 1)
    half = (col // 2).astype(jnp.float32)
    angle = pos * jnp.exp(half * (-2.0 * math.log(10000.0) / emb))
    odd = (col % 2) == 1
    out_ref[...] = jnp.where(odd, jnp.cos(angle), jnp.sin(angle))


def kernel(inputs, table):
    batch, hist = inputs.shape
    n = batch * hist
    idx2d = inputs.astype(jnp.int32).reshape(n // GROUP, GROUP)
    flat = _gather_call(n)(idx2d, table)
    pe = pl.pallas_call(
        _pe_body,
        out_shape=jax.ShapeDtypeStruct((hist, EMBED), jnp.float32),
    )()
    return flat.reshape(batch, hist, EMBED), pe


# speedup vs baseline: 1.0131x; 1.0131x over previous
import functools
import math

import jax
import jax.numpy as jnp
from jax import lax
from jax.experimental import pallas as pl
from jax.experimental.pallas import tpu as pltpu
from jax.experimental.pallas import tpu_sc as plsc

EMBED = 64
NC, NS = 2, 16
NW = NC * NS
GROUP = 128
K = 5
CHUNK = GROUP * K


@functools.lru_cache(maxsize=None)
def _gather_call(n_total):
    per_w = n_total // NW
    n_grp = per_w // GROUP
    n_chunks = per_w // CHUNK
    mesh = plsc.VectorSubcoreMesh(core_axis_name="c", subcore_axis_name="s")

    @functools.partial(
        pl.kernel,
        out_type=jax.ShapeDtypeStruct((n_total, EMBED), jnp.float32),
        mesh=mesh,
        scratch_types=[
            pltpu.VMEM((n_grp, GROUP), jnp.int32),
            pltpu.VMEM((CHUNK, EMBED), jnp.float32),
            pltpu.VMEM((CHUNK, EMBED), jnp.float32),
            pltpu.SemaphoreType.DMA,
            pltpu.SemaphoreType.DMA,
            pltpu.SemaphoreType.DMA,
            pltpu.SemaphoreType.DMA,
        ],
        compiler_params=pltpu.CompilerParams(use_tc_tiling_on_sc=False),
    )
    def gather(idx_hbm, table_hbm, out_hbm, idx_all, rows0, rows1,
               gsem0, gsem1, wsem0, wsem1):
        wid = lax.axis_index("s") * NC + lax.axis_index("c")
        grp0 = wid * n_grp
        row0 = wid * per_w

        pltpu.sync_copy(idx_hbm.at[pl.ds(grp0, n_grp)], idx_all)

        def fire(c, rows, sem):
            for j in range(K):
                pltpu.async_copy(
                    table_hbm.at[idx_all.at[c * K + j]],
                    rows.at[pl.ds(j * GROUP, GROUP)],
                    sem,
                )

        def gwait(rows, sem):
            pltpu.make_async_copy(out_hbm.at[pl.ds(row0, CHUNK)], rows, sem).wait()

        def wfire(c, rows, sem):
            pltpu.async_copy(rows, out_hbm.at[pl.ds(row0 + c * CHUNK, CHUNK)], sem)

        def wwait(rows, sem):
            pltpu.make_async_copy(rows, out_hbm.at[pl.ds(row0, CHUNK)], sem).wait()

        fire(0, rows0, gsem0)
        fire(1, rows1, gsem1)

        @pl.loop(0, n_chunks - 2, step=2)
        def _pair(g):
            gwait(rows0, gsem0)
            wfire(g, rows0, wsem0)
            gwait(rows1, gsem1)
            wfire(g + 1, rows1, wsem1)
            wwait(rows0, wsem0)
            fire(g + 2, rows0, gsem0)
            wwait(rows1, wsem1)
            fire(g + 3, rows1, gsem1)

        gwait(rows0, gsem0)
        wfire(n_chunks - 2, rows0, wsem0)
        gwait(rows1, gsem1)
        wfire(n_chunks - 1, rows1, wsem1)
        wwait(rows0, wsem0)
        wwait(rows1, wsem1)

    return gather


def _pe_body(out_ref):
    hist, emb = out_ref.shape
    pos = lax.broadcasted_iota(jnp.int32, (hist, emb), 0).astype(jnp.float32)
    col = lax.broadcasted_iota(jnp.int32, (hist, emb), 1)
    half = (col // 2).astype(jnp.float32)
    angle = pos * jnp.exp(half * (-2.0 * math.log(10000.0) / emb))
    odd = (col % 2) == 1
    out_ref[...] = jnp.where(odd, jnp.cos(angle), jnp.sin(angle))


def kernel(inputs, table):
    batch, hist = inputs.shape
    n = batch * hist
    idx2d = inputs.astype(jnp.int32).reshape(n // GROUP, GROUP)
    flat = _gather_call(n)(idx2d, table)
    pe = pl.pallas_call(
        _pe_body,
        out_shape=jax.ShapeDtypeStruct((hist, EMBED), jnp.float32),
    )()
    return flat.reshape(batch, hist, EMBED), pe
